# Initial kernel scaffold; baseline (speedup 1.0000x reference)
import jax
import jax.numpy as jnp
from jax.experimental import pallas as pl


def kernel(hidden_states, router_w, gate_up_proj, down_proj, shared_gate_w, shared_up_w, shared_down_w):
    raise NotImplementedError("write your pallas kernel here")


# speedup vs baseline: 1.4766x; 1.4766x over previous
import functools

import jax
import jax.numpy as jnp
from jax.experimental import pallas as pl
import jax.experimental.pallas.tpu as pltpu

E = 16
TOPK = 2
H = 1024
FF = 2048
T = 128

FB = 1024
FFB = FF // FB


def _router_kernel(hs_ref, rw_ref, scores_ref):
    logits = jax.lax.dot_general(
        hs_ref[...], rw_ref[...], (((1,), (1,)), ((), ())),
        preferred_element_type=jnp.float32)
    iota_e = jax.lax.broadcasted_iota(jnp.int32, (T, E), 1)
    m1 = jnp.max(logits, axis=1, keepdims=True)
    pos1 = jnp.min(jnp.where(logits == m1, iota_e, E), axis=1, keepdims=True)
    oh1 = iota_e == pos1
    masked = jnp.where(oh1, -jnp.inf, logits)
    m2 = jnp.max(masked, axis=1, keepdims=True)
    pos2 = jnp.min(jnp.where(masked == m2, iota_e, E), axis=1, keepdims=True)
    oh2 = iota_e == pos2
    sel = jnp.logical_or(oh1, oh2)
    scores_te = jnp.where(sel, jax.nn.sigmoid(logits), 0.0)
    scores_ref[...] = scores_te.T


def _shared_kernel(hs_ref, sg_ref, su_ref, sd_ref, out_ref):
    hs = hs_ref[...]
    g = jax.lax.dot_general(hs, sg_ref[...], (((1,), (1,)), ((), ())),
                            preferred_element_type=jnp.float32)
    u = jax.lax.dot_general(hs, su_ref[...], (((1,), (1,)), ((), ())),
                            preferred_element_type=jnp.float32)
    act = jax.nn.silu(g) * u
    out_ref[...] = jax.lax.dot_general(act, sd_ref[...], (((1,), (1,)), ((), ())),
                                       preferred_element_type=jnp.float32)


def _expert_kernel(scores_t_ref, hs_ref, gate_ref, up_ref, down_ref,
                   shared_ref, out_ref):
    e = pl.program_id(0)
    f = pl.program_id(1)
    iota_e = jax.lax.broadcasted_iota(jnp.int32, (T, E), 1)
    sc = jnp.sum(jnp.where(iota_e == e, scores_t_ref[...], 0.0),
                 axis=1, keepdims=True)
    x = hs_ref[...] * sc
    g = jax.lax.dot_general(x, gate_ref[0], (((1,), (0,)), ((), ())),
                            preferred_element_type=jnp.float32)
    u = jax.lax.dot_general(x, up_ref[0], (((1,), (0,)), ((), ())),
                            preferred_element_type=jnp.float32)
    act = jax.nn.silu(g) * u
    part = jax.lax.dot_general(act, down_ref[0], (((1,), (0,)), ((), ())),
                               preferred_element_type=jnp.float32)
    first = jnp.logical_and(e == 0, f == 0)

    @pl.when(first)
    def _():
        out_ref[...] = shared_ref[...] + part

    @pl.when(jnp.logical_not(first))
    def _():
        out_ref[...] += part


@jax.jit
def kernel(hidden_states, router_w, gate_up_proj, down_proj,
           shared_gate_w, shared_up_w, shared_down_w):
    hs = hidden_states.reshape(-1, H)

    router_scores = pl.pallas_call(
        _router_kernel,
        out_shape=jax.ShapeDtypeStruct((E, T), jnp.float32),
    )(hs, router_w)

    shared_out = pl.pallas_call(
        _shared_kernel,
        out_shape=jax.ShapeDtypeStruct((T, H), jnp.float32),
    )(hs, shared_gate_w, shared_up_w, shared_down_w)

    out = pl.pallas_call(
        _expert_kernel,
        grid=(E, FFB),
        in_specs=[
            pl.BlockSpec((T, E), lambda e, f: (0, 0)),
            pl.BlockSpec((T, H), lambda e, f: (0, 0)),
            pl.BlockSpec((1, H, FB), lambda e, f: (e, 0, f)),
            pl.BlockSpec((1, H, FB), lambda e, f: (e, 0, f + FFB)),
            pl.BlockSpec((1, FB, H), lambda e, f: (e, f, 0)),
            pl.BlockSpec((T, H), lambda e, f: (0, 0)),
        ],
        out_specs=pl.BlockSpec((T, H), lambda e, f: (0, 0)),
        out_shape=jax.ShapeDtypeStruct((T, H), jnp.float32),
        compiler_params=pltpu.CompilerParams(
            dimension_semantics=("arbitrary", "arbitrary")),
    )(router_scores.T, hs, gate_up_proj, gate_up_proj, down_proj, shared_out)

    return (out, router_scores)


# speedup vs baseline: 1.4818x; 1.0035x over previous
import functools

import jax
import jax.numpy as jnp
from jax.experimental import pallas as pl
import jax.experimental.pallas.tpu as pltpu

E = 16
TOPK = 2
H = 1024
FF = 2048
T = 128

FB = 1024
FFB = FF // FB


def _router_kernel(hs_ref, rw_ref, scores_ref):
    logits = jax.lax.dot_general(
        hs_ref[...], rw_ref[...], (((1,), (1,)), ((), ())),
        preferred_element_type=jnp.float32)
    iota_e = jax.lax.broadcasted_iota(jnp.int32, (T, E), 1)
    m1 = jnp.max(logits, axis=1, keepdims=True)
    pos1 = jnp.min(jnp.where(logits == m1, iota_e, E), axis=1, keepdims=True)
    oh1 = iota_e == pos1
    masked = jnp.where(oh1, -jnp.inf, logits)
    m2 = jnp.max(masked, axis=1, keepdims=True)
    pos2 = jnp.min(jnp.where(masked == m2, iota_e, E), axis=1, keepdims=True)
    oh2 = iota_e == pos2
    sel = jnp.logical_or(oh1, oh2)
    scores_te = jnp.where(sel, jax.nn.sigmoid(logits), 0.0)
    scores_ref[...] = scores_te.T


def _shared_kernel(hs_ref, sg_ref, su_ref, sd_ref, out_ref):
    hs = hs_ref[...]
    g = jax.lax.dot_general(hs, sg_ref[...], (((1,), (1,)), ((), ())),
                            preferred_element_type=jnp.float32)
    u = jax.lax.dot_general(hs, su_ref[...], (((1,), (1,)), ((), ())),
                            preferred_element_type=jnp.float32)
    act = jax.nn.silu(g) * u
    out_ref[...] = jax.lax.dot_general(act, sd_ref[...], (((1,), (1,)), ((), ())),
                                       preferred_element_type=jnp.float32)


def _expert_kernel(scores_t_ref, hs_ref, gate_ref, up_ref, down_ref,
                   shared_ref, out_ref):
    e = pl.program_id(0)
    f = pl.program_id(1)
    iota_e = jax.lax.broadcasted_iota(jnp.int32, (T, E), 1)
    sc = jnp.sum(jnp.where(iota_e == e, scores_t_ref[...], 0.0),
                 axis=1, keepdims=True)
    x = (hs_ref[...] * sc).astype(jnp.bfloat16)
    g = jax.lax.dot_general(x, gate_ref[0].astype(jnp.bfloat16),
                            (((1,), (0,)), ((), ())),
                            preferred_element_type=jnp.float32)
    u = jax.lax.dot_general(x, up_ref[0].astype(jnp.bfloat16),
                            (((1,), (0,)), ((), ())),
                            preferred_element_type=jnp.float32)
    act = (jax.nn.silu(g) * u).astype(jnp.bfloat16)
    part = jax.lax.dot_general(act, down_ref[0].astype(jnp.bfloat16),
                               (((1,), (0,)), ((), ())),
                               preferred_element_type=jnp.float32)
    first = jnp.logical_and(e == 0, f == 0)

    @pl.when(first)
    def _():
        out_ref[...] = shared_ref[...] + part

    @pl.when(jnp.logical_not(first))
    def _():
        out_ref[...] += part


@jax.jit
def kernel(hidden_states, router_w, gate_up_proj, down_proj,
           shared_gate_w, shared_up_w, shared_down_w):
    hs = hidden_states.reshape(-1, H)

    router_scores = pl.pallas_call(
        _router_kernel,
        out_shape=jax.ShapeDtypeStruct((E, T), jnp.float32),
    )(hs, router_w)

    shared_out = pl.pallas_call(
        _shared_kernel,
        out_shape=jax.ShapeDtypeStruct((T, H), jnp.float32),
    )(hs, shared_gate_w, shared_up_w, shared_down_w)

    out = pl.pallas_call(
        _expert_kernel,
        grid=(E, FFB),
        in_specs=[
            pl.BlockSpec((T, E), lambda e, f: (0, 0)),
            pl.BlockSpec((T, H), lambda e, f: (0, 0)),
            pl.BlockSpec((1, H, FB), lambda e, f: (e, 0, f)),
            pl.BlockSpec((1, H, FB), lambda e, f: (e, 0, f + FFB)),
            pl.BlockSpec((1, FB, H), lambda e, f: (e, f, 0)),
            pl.BlockSpec((T, H), lambda e, f: (0, 0)),
        ],
        out_specs=pl.BlockSpec((T, H), lambda e, f: (0, 0)),
        out_shape=jax.ShapeDtypeStruct((T, H), jnp.float32),
        compiler_params=pltpu.CompilerParams(
            dimension_semantics=("arbitrary", "arbitrary")),
    )(router_scores.T, hs, gate_up_proj, gate_up_proj, down_proj, shared_out)

    return (out, router_scores)


# speedup vs baseline: 1.5477x; 1.0445x over previous
import jax
import jax.numpy as jnp
from jax.experimental import pallas as pl
import jax.experimental.pallas.tpu as pltpu

E = 16
TOPK = 2
H = 1024
FF = 2048
T = 128

FB = 1024
FFB = FF // FB


def _moe_kernel(hs_ref, rw_ref, gate_ref, up_ref, down_ref,
                sg_ref, su_ref, sd_ref,
                out_ref, scores_out_ref, scores_scr, act_scr):
    g = pl.program_id(0)

    @pl.when(g == 0)
    def _():
        hs = hs_ref[...]
        logits = jax.lax.dot_general(
            hs, rw_ref[...], (((1,), (1,)), ((), ())),
            preferred_element_type=jnp.float32)
        iota_e = jax.lax.broadcasted_iota(jnp.int32, (T, E), 1)
        m1 = jnp.max(logits, axis=1, keepdims=True)
        pos1 = jnp.min(jnp.where(logits == m1, iota_e, E), axis=1,
                       keepdims=True)
        oh1 = iota_e == pos1
        masked = jnp.where(oh1, -jnp.inf, logits)
        m2 = jnp.max(masked, axis=1, keepdims=True)
        pos2 = jnp.min(jnp.where(masked == m2, iota_e, E), axis=1,
                       keepdims=True)
        oh2 = iota_e == pos2
        sel = jnp.logical_or(oh1, oh2)
        scores_te = jnp.where(sel, jax.nn.sigmoid(logits), 0.0)
        scores_scr[...] = scores_te
        scores_out_ref[...] = scores_te.T
        hsb = hs.astype(jnp.bfloat16)
        gsh = jax.lax.dot_general(hsb, sg_ref[...].astype(jnp.bfloat16),
                                  (((1,), (1,)), ((), ())),
                                  preferred_element_type=jnp.float32)
        ush = jax.lax.dot_general(hsb, su_ref[...].astype(jnp.bfloat16),
                                  (((1,), (1,)), ((), ())),
                                  preferred_element_type=jnp.float32)
        act_scr[...] = jax.nn.silu(gsh) * ush

    @pl.when(g == 1)
    def _():
        out_ref[...] = jax.lax.dot_general(
            act_scr[...].astype(jnp.bfloat16),
            sd_ref[...].astype(jnp.bfloat16),
            (((1,), (1,)), ((), ())),
            preferred_element_type=jnp.float32)

    @pl.when(g >= 2)
    def _():
        e = (g - 2) // FFB
        iota_e = jax.lax.broadcasted_iota(jnp.int32, (T, E), 1)
        sc = jnp.sum(jnp.where(iota_e == e, scores_scr[...], 0.0),
                     axis=1, keepdims=True)
        x = (hs_ref[...] * sc).astype(jnp.bfloat16)
        gmat = jax.lax.dot_general(x, gate_ref[0].astype(jnp.bfloat16),
                                   (((1,), (0,)), ((), ())),
                                   preferred_element_type=jnp.float32)
        umat = jax.lax.dot_general(x, up_ref[0].astype(jnp.bfloat16),
                                   (((1,), (0,)), ((), ())),
                                   preferred_element_type=jnp.float32)
        act = (jax.nn.silu(gmat) * umat).astype(jnp.bfloat16)
        out_ref[...] += jax.lax.dot_general(
            act, down_ref[0].astype(jnp.bfloat16),
            (((1,), (0,)), ((), ())),
            preferred_element_type=jnp.float32)


def _e_idx(g):
    return jnp.maximum(g - 2, 0) // FFB


def _f_idx(g):
    return jnp.maximum(g - 2, 0) % FFB


@jax.jit
def kernel(hidden_states, router_w, gate_up_proj, down_proj,
           shared_gate_w, shared_up_w, shared_down_w):
    hs = hidden_states.reshape(-1, H)

    out, router_scores = pl.pallas_call(
        _moe_kernel,
        grid=(2 + E * FFB,),
        in_specs=[
            pl.BlockSpec((T, H), lambda g: (0, 0)),
            pl.BlockSpec((E, H), lambda g: (0, 0)),
            pl.BlockSpec((1, H, FB), lambda g: (_e_idx(g), 0, _f_idx(g))),
            pl.BlockSpec((1, H, FB),
                         lambda g: (_e_idx(g), 0, _f_idx(g) + FFB)),
            pl.BlockSpec((1, FB, H), lambda g: (_e_idx(g), _f_idx(g), 0)),
            pl.BlockSpec((FF, H), lambda g: (0, 0)),
            pl.BlockSpec((FF, H), lambda g: (0, 0)),
            pl.BlockSpec((H, FF), lambda g: (0, 0)),
        ],
        out_specs=[
            pl.BlockSpec((T, H), lambda g: (0, 0)),
            pl.BlockSpec((E, T), lambda g: (0, 0)),
        ],
        out_shape=[
            jax.ShapeDtypeStruct((T, H), jnp.float32),
            jax.ShapeDtypeStruct((E, T), jnp.float32),
        ],
        scratch_shapes=[
            pltpu.VMEM((T, E), jnp.float32),
            pltpu.VMEM((T, FF), jnp.float32),
        ],
        compiler_params=pltpu.CompilerParams(
            dimension_semantics=("arbitrary",),
            vmem_limit_bytes=60 * 1024 * 1024,
        ),
    )(hs, router_w, gate_up_proj, gate_up_proj, down_proj,
      shared_gate_w, shared_up_w, shared_down_w)

    return (out, router_scores)
